# baseline (speedup 1.0000x reference)
import functools

import jax
import jax.numpy as jnp
from jax import lax
from jax.experimental import pallas as pl
from jax.experimental.pallas import tpu as pltpu
from jax.experimental.pallas import tpu_sc as plsc

_NUM_CORES = 2
_NUM_SUBCORES = 16
_NW = _NUM_CORES * _NUM_SUBCORES

_BATCH = 16384
_DIM = 32
_BPW = _BATCH // _NW


@functools.partial(
    pl.kernel,
    out_type=jax.ShapeDtypeStruct((_BATCH, _DIM), jnp.float32),
    mesh=plsc.VectorSubcoreMesh(
        core_axis_name="c",
        subcore_axis_name="s",
        num_cores=_NUM_CORES,
        num_subcores=_NUM_SUBCORES,
    ),
    scratch_types=[
        pltpu.VMEM((_BPW,), jnp.int32),
        pltpu.VMEM((_BPW, _DIM), jnp.float32),
        pltpu.SemaphoreType.DMA,
    ],
    compiler_params=pltpu.CompilerParams(use_tc_tiling_on_sc=False),
)
def _gather_kernel(idx_hbm, table_hbm, out_hbm, idx_v, rows_v, sem):
    wid = lax.axis_index("s") * _NUM_CORES + lax.axis_index("c")
    base = wid * _BPW
    pltpu.sync_copy(idx_hbm.at[pl.ds(base, _BPW)], idx_v)
    pltpu.async_copy(table_hbm.at[idx_v], rows_v, sem).wait()
    pltpu.sync_copy(rows_v, out_hbm.at[pl.ds(base, _BPW)])


def kernel(zone_idx, embedding_table):
    return _gather_kernel(zone_idx.astype(jnp.int32), embedding_table)


# speedup vs baseline: 1.9577x; 1.9577x over previous
import functools

import jax
import jax.numpy as jnp
from jax import lax
from jax.experimental import pallas as pl
from jax.experimental.pallas import tpu as pltpu
from jax.experimental.pallas import tpu_sc as plsc

_NUM_CORES = 2
_NUM_SUBCORES = 16
_NW = _NUM_CORES * _NUM_SUBCORES

_BATCH = 16384
_DIM = 32
_ZONES = 9
_BPW = _BATCH // _NW
_CHUNKS = _BPW // 16


@functools.partial(
    pl.kernel,
    out_type=jax.ShapeDtypeStruct((_BATCH * _DIM,), jnp.float32),
    mesh=plsc.VectorSubcoreMesh(
        core_axis_name="c",
        subcore_axis_name="s",
        num_cores=_NUM_CORES,
        num_subcores=_NUM_SUBCORES,
    ),
    scratch_types=[
        pltpu.VMEM((_BPW,), jnp.int32),
        pltpu.VMEM((_ZONES * _DIM,), jnp.float32),
        pltpu.VMEM((_BPW * _DIM,), jnp.float32),
    ],
    compiler_params=pltpu.CompilerParams(
        use_tc_tiling_on_sc=False, needs_layout_passes=False
    ),
)
def _gather_kernel(idx_hbm, table_hbm, out_hbm, idx_v, table_v, rows_v):
    wid = lax.axis_index("s") * _NUM_CORES + lax.axis_index("c")
    base = wid * _BPW
    pltpu.sync_copy(idx_hbm.at[pl.ds(base, _BPW)], idx_v)
    pltpu.sync_copy(table_hbm, table_v)
    col = lax.iota(jnp.int32, 16) * _DIM

    def body(i, carry):
        rowbase = idx_v[pl.ds(i * 16, 16)] * _DIM
        outbase = col + i * (16 * _DIM)
        for d in range(_DIM):
            vals = plsc.load_gather(table_v, [rowbase + d])
            plsc.store_scatter(rows_v, [outbase + d], vals)
        return carry

    lax.fori_loop(0, _CHUNKS, body, 0)
    pltpu.sync_copy(rows_v, out_hbm.at[pl.ds(base * _DIM, _BPW * _DIM)])


def kernel(zone_idx, embedding_table):
    out = _gather_kernel(zone_idx.astype(jnp.int32), embedding_table.reshape(-1))
    return out.reshape(_BATCH, _DIM)


# speedup vs baseline: 2.1382x; 1.0922x over previous
import functools

import jax
import jax.numpy as jnp
from jax import lax
from jax.experimental import pallas as pl
from jax.experimental.pallas import tpu as pltpu
from jax.experimental.pallas import tpu_sc as plsc

_NUM_CORES = 2
_NUM_SUBCORES = 16
_NW = _NUM_CORES * _NUM_SUBCORES

_BATCH = 16384
_DIM = 32
_ZONES = 9
_BPW = _BATCH // _NW
_CHUNKS = _BPW // 16


@functools.partial(
    pl.kernel,
    out_type=jax.ShapeDtypeStruct((_BATCH * _DIM,), jnp.float32),
    mesh=plsc.VectorSubcoreMesh(
        core_axis_name="c",
        subcore_axis_name="s",
        num_cores=_NUM_CORES,
        num_subcores=_NUM_SUBCORES,
    ),
    scratch_types=[
        pltpu.VMEM((_BPW,), jnp.int32),
        pltpu.VMEM((_ZONES * _DIM,), jnp.float32),
        pltpu.VMEM((_BPW * _DIM,), jnp.float32),
    ],
    compiler_params=pltpu.CompilerParams(
        use_tc_tiling_on_sc=False, needs_layout_passes=False
    ),
)
def _gather_kernel(idx_hbm, table_hbm, out_hbm, idx_v, table_v, rows_v):
    wid = lax.axis_index("s") * _NUM_CORES + lax.axis_index("c")
    base = wid * _BPW
    pltpu.sync_copy(idx_hbm.at[pl.ds(base, _BPW)], idx_v)
    pltpu.sync_copy(table_hbm, table_v)
    col = lax.iota(jnp.int32, 16) * _DIM

    @plsc.parallel_loop(0, _CHUNKS, 1, unroll=2)
    def body(i):
        rowbase = idx_v[pl.ds(i * 16, 16)] * _DIM
        outbase = col + i * (16 * _DIM)
        for d in range(_DIM):
            vals = plsc.load_gather(table_v, [rowbase + d])
            plsc.store_scatter(rows_v, [outbase + d], vals)
    pltpu.sync_copy(rows_v, out_hbm.at[pl.ds(base * _DIM, _BPW * _DIM)])


def kernel(zone_idx, embedding_table):
    out = _gather_kernel(zone_idx.astype(jnp.int32), embedding_table.reshape(-1))
    return out.reshape(_BATCH, _DIM)


# speedup vs baseline: 2.4165x; 1.1302x over previous
import functools

import jax
import jax.numpy as jnp
from jax import lax
from jax.experimental import pallas as pl
from jax.experimental.pallas import tpu as pltpu
from jax.experimental.pallas import tpu_sc as plsc

_NUM_CORES = 2
_NUM_SUBCORES = 16
_NW = _NUM_CORES * _NUM_SUBCORES

_BATCH = 16384
_DIM = 32
_ZONES = 9
_BPW = _BATCH // _NW
_CHUNKS = _BPW // 16


@functools.partial(
    pl.kernel,
    out_type=jax.ShapeDtypeStruct((_BATCH, _DIM), jnp.float32),
    mesh=plsc.VectorSubcoreMesh(
        core_axis_name="c",
        subcore_axis_name="s",
        num_cores=_NUM_CORES,
        num_subcores=_NUM_SUBCORES,
    ),
    scratch_types=[
        pltpu.VMEM((_BPW,), jnp.int32),
        pltpu.VMEM((_ZONES, _DIM), jnp.float32),
        pltpu.VMEM((_BPW, _DIM), jnp.float32),
    ],
    compiler_params=pltpu.CompilerParams(needs_layout_passes=False),
)
def _gather_kernel(idx_hbm, table_hbm, out_hbm, idx_v, table_v, rows_v):
    wid = lax.axis_index("s") * _NUM_CORES + lax.axis_index("c")
    base = wid * _BPW
    pltpu.sync_copy(idx_hbm.at[pl.ds(base, _BPW)], idx_v)
    pltpu.sync_copy(table_hbm, table_v)
    row16 = lax.iota(jnp.int32, 16)

    @plsc.parallel_loop(0, _CHUNKS, 1, unroll=2)
    def body(i):
        rowidx = idx_v[pl.ds(i * 16, 16)]
        outrow = row16 + i * 16
        for d in range(_DIM):
            dvec = jnp.full((16,), d, dtype=jnp.int32)
            vals = plsc.load_gather(table_v, [rowidx, dvec])
            plsc.store_scatter(rows_v, [outrow, dvec], vals)

    pltpu.sync_copy(rows_v, out_hbm.at[pl.ds(base, _BPW)])


def kernel(zone_idx, embedding_table):
    return _gather_kernel(zone_idx.astype(jnp.int32), embedding_table)


# speedup vs baseline: 3.6022x; 1.4906x over previous
import functools

import jax
import jax.numpy as jnp
from jax import lax
from jax.experimental import pallas as pl
from jax.experimental.pallas import tpu as pltpu
from jax.experimental.pallas import tpu_sc as plsc

_NUM_CORES = 2
_NUM_SUBCORES = 16
_NW = _NUM_CORES * _NUM_SUBCORES

_BATCH = 16384
_DIM = 32
_ZONES = 9
_BPW = _BATCH // _NW
_CHUNKS = _BPW // 16


@functools.partial(
    pl.kernel,
    out_type=jax.ShapeDtypeStruct((_DIM, _BATCH), jnp.float32),
    mesh=plsc.VectorSubcoreMesh(
        core_axis_name="c",
        subcore_axis_name="s",
        num_cores=_NUM_CORES,
        num_subcores=_NUM_SUBCORES,
    ),
    scratch_types=[
        pltpu.VMEM((_BPW,), jnp.int32),
        pltpu.VMEM((_ZONES, _DIM), jnp.float32),
        pltpu.VMEM((_DIM, _BPW), jnp.float32),
    ],
    compiler_params=pltpu.CompilerParams(needs_layout_passes=False),
)
def _gather_kernel(idx_hbm, table_hbm, out_hbm, idx_v, table_v, rows_v):
    wid = lax.axis_index("s") * _NUM_CORES + lax.axis_index("c")
    base = wid * _BPW
    pltpu.sync_copy(idx_hbm.at[pl.ds(base, _BPW)], idx_v)
    pltpu.sync_copy(table_hbm, table_v)

    @plsc.parallel_loop(0, _CHUNKS, 1, unroll=2)
    def body(i):
        rowidx = idx_v[pl.ds(i * 16, 16)]
        for d in range(_DIM):
            dvec = jnp.full((16,), d, dtype=jnp.int32)
            vals = plsc.load_gather(table_v, [rowidx, dvec])
            rows_v[d, pl.ds(i * 16, 16)] = vals

    pltpu.sync_copy(rows_v, out_hbm.at[:, pl.ds(base, _BPW)])


def kernel(zone_idx, embedding_table):
    out_t = _gather_kernel(zone_idx.astype(jnp.int32), embedding_table)
    return out_t.T
